# baseline (speedup 1.0000x reference)
import functools

import jax
import jax.numpy as jnp
from jax import lax
from jax.experimental import pallas as pl
from jax.experimental.pallas import tpu as pltpu
from jax.experimental.pallas import tpu_sc as plsc

OP = 3


def _sc_gather(table, idx):
    info = plsc.get_sparse_core_info()
    nw = info.num_cores * info.num_subcores
    n, d = idx.shape[0], table.shape[1]
    n_per_w = n // nw
    ch = 64
    n_ch = n_per_w // ch
    nbuf = 3
    mesh = plsc.VectorSubcoreMesh(core_axis_name="c", subcore_axis_name="s")

    @functools.partial(
        pl.kernel,
        mesh=mesh,
        out_type=jax.ShapeDtypeStruct((n, d), jnp.float32),
        scratch_types=[pltpu.VMEM((n_per_w,), jnp.int32)]
        + [pltpu.VMEM((ch, d), jnp.float32)] * nbuf
        + [pltpu.SemaphoreType.DMA] * (2 * nbuf),
    )
    def k(table_hbm, idx_hbm, out_hbm, idx_v, *bufsem):
        bufs = bufsem[:nbuf]
        gsems = bufsem[nbuf:2 * nbuf]
        wsems = bufsem[2 * nbuf:]
        wid = lax.axis_index("s") * info.num_cores + lax.axis_index("c")
        base = wid * n_per_w
        pltpu.sync_copy(idx_hbm.at[pl.ds(base, n_per_w)], idx_v)

        def start_g(c):
            return pltpu.async_copy(
                table_hbm.at[idx_v.at[pl.ds(c * ch, ch)]],
                bufs[c % nbuf], gsems[c % nbuf])

        gh = {c: start_g(c) for c in range(min(nbuf, n_ch))}
        for c in range(n_ch):
            gh[c].wait()
            wbh = pltpu.async_copy(
                bufs[c % nbuf], out_hbm.at[pl.ds(base + c * ch, ch)],
                wsems[c % nbuf])
            wbh.wait()
            if c + nbuf < n_ch:
                gh[c + nbuf] = start_g(c + nbuf)

    return k(table, idx)


def _fold_body(syn_W2, com_W2, out_Wp, syn_b2, com_b2, out_bp,
               fold_s, fold_c, cbias):
    d = syn_W2.shape[1]
    wo_s = out_Wp[:d, :]
    wo_c = out_Wp[d:, :]
    fold_s[...] = jnp.dot(syn_W2[...], wo_s, preferred_element_type=jnp.float32)
    fold_c[...] = jnp.dot(com_W2[...], wo_c, preferred_element_type=jnp.float32)
    cbias[...] = (jnp.dot(syn_b2[...], wo_s, preferred_element_type=jnp.float32)
                  + jnp.dot(com_b2[...], wo_c, preferred_element_type=jnp.float32)
                  + out_bp[...])


SB = 16


def _main_body(x_ref, a1_ref, a2_ref, w1s_ref, w1c_ref, b1s_ref, b1c_ref,
               folds_ref, foldc_ref, cbias_ref, out_ref):
    bf = jnp.bfloat16
    l, d = x_ref.shape[1], x_ref.shape[2]
    x2 = x_ref[...].reshape(SB * l, d).astype(bf)
    ps2 = jnp.dot(x2, w1s_ref[...],
                  preferred_element_type=jnp.float32).astype(bf)
    pc2 = jnp.dot(x2, w1c_ref[...],
                  preferred_element_type=jnp.float32).astype(bf)
    for s in range(SB):
        a1 = a1_ref[s].astype(bf)
        a2 = a2_ref[s].astype(bf)
        ps = ps2[s * l:(s + 1) * l]
        pc = pc2[s * l:(s + 1) * l]
        hs = jnp.maximum(
            jnp.dot(a1, ps, preferred_element_type=jnp.float32)
            + b1s_ref[...], 0.0)
        hc = jnp.maximum(
            jnp.dot(a2, pc, preferred_element_type=jnp.float32)
            + b1c_ref[...], 0.0)
        qs = jnp.dot(hs.astype(bf), folds_ref[...],
                     preferred_element_type=jnp.float32)
        qc = jnp.dot(hc.astype(bf), foldc_ref[...],
                     preferred_element_type=jnp.float32)
        out_ref[s] = (
            jnp.dot(a1, qs.astype(bf), preferred_element_type=jnp.float32)
            + jnp.dot(a2, qc.astype(bf), preferred_element_type=jnp.float32)
            + cbias_ref[...])


def kernel(inputs, adj1, adj2, adj3, table, syn_W1, syn_b1, syn_W2, syn_b2,
           com_W1, com_b1, com_W2, com_b2, out_W, out_b):
    b, l = inputs.shape
    d, h = syn_W1.shape
    o = out_W.shape[1]

    idx = inputs.reshape(b * l).astype(jnp.int32)
    nchunk = 1
    bc = b // nchunk
    xs = [_sc_gather(table, idx[k * bc * l:(k + 1) * bc * l]).reshape(bc, l, d)
          for k in range(nchunk)]

    out_Wp = jnp.zeros((2 * d, OP), jnp.float32).at[:, :o].set(out_W)
    out_bp = jnp.zeros((1, OP), jnp.float32).at[:, :o].set(out_b)

    fold_s, fold_c, cbias = pl.pallas_call(
        _fold_body,
        out_shape=[
            jax.ShapeDtypeStruct((h, OP), jnp.float32),
            jax.ShapeDtypeStruct((h, OP), jnp.float32),
            jax.ShapeDtypeStruct((1, OP), jnp.float32),
        ],
    )(syn_W2, com_W2, out_Wp, syn_b2.reshape(1, d), com_b2.reshape(1, d),
      out_bp)

    full = lambda shape: pl.BlockSpec(shape, lambda i: (0,) * len(shape))
    main = pl.pallas_call(
        _main_body,
        grid=(bc // SB,),
        in_specs=[
            pl.BlockSpec((SB, l, d), lambda i: (i, 0, 0)),
            pl.BlockSpec((SB, l, l), lambda i: (i, 0, 0)),
            pl.BlockSpec((SB, l, l), lambda i: (i, 0, 0)),
            full((d, h)),
            full((d, h)),
            full((1, h)),
            full((1, h)),
            full((h, OP)),
            full((h, OP)),
            full((1, OP)),
        ],
        out_specs=pl.BlockSpec((SB, l, OP), lambda i: (i, 0, 0)),
        out_shape=jax.ShapeDtypeStruct((bc, l, OP), jnp.float32),
        compiler_params=pltpu.CompilerParams(
            dimension_semantics=("arbitrary",)),
    )
    w1s = syn_W1.astype(jnp.bfloat16)
    w1c = com_W1.astype(jnp.bfloat16)
    fs = fold_s.astype(jnp.bfloat16)
    fc = fold_c.astype(jnp.bfloat16)
    outs = [
        main(xs[k], adj1[k * bc:(k + 1) * bc], adj2[k * bc:(k + 1) * bc],
             w1s, w1c, syn_b1.reshape(1, h), com_b1.reshape(1, h),
             fs, fc, cbias)
        for k in range(nchunk)
    ]
    return jnp.concatenate(outs, axis=0)[:, :, :o]


# speedup vs baseline: 1.0025x; 1.0025x over previous
import functools

import jax
import jax.numpy as jnp
from jax import lax
from jax.experimental import pallas as pl
from jax.experimental.pallas import tpu as pltpu
from jax.experimental.pallas import tpu_sc as plsc

OP = 3


def _sc_gather(table, idx):
    info = plsc.get_sparse_core_info()
    nw = info.num_cores * info.num_subcores
    n, d = idx.shape[0], table.shape[1]
    n_per_w = n // nw
    ch = 64
    n_ch = n_per_w // ch
    nbuf = 3
    mesh = plsc.VectorSubcoreMesh(core_axis_name="c", subcore_axis_name="s")

    @functools.partial(
        pl.kernel,
        mesh=mesh,
        out_type=jax.ShapeDtypeStruct((n, d), jnp.float32),
        scratch_types=[pltpu.VMEM((n_per_w,), jnp.int32)]
        + [pltpu.VMEM((ch, d), jnp.float32)] * nbuf
        + [pltpu.SemaphoreType.DMA] * (2 * nbuf),
    )
    def k(table_hbm, idx_hbm, out_hbm, idx_v, *bufsem):
        bufs = bufsem[:nbuf]
        gsems = bufsem[nbuf:2 * nbuf]
        wsems = bufsem[2 * nbuf:]
        wid = lax.axis_index("s") * info.num_cores + lax.axis_index("c")
        base = wid * n_per_w
        pltpu.sync_copy(idx_hbm.at[pl.ds(base, n_per_w)], idx_v)

        def start_g(c):
            return pltpu.async_copy(
                table_hbm.at[idx_v.at[pl.ds(c * ch, ch)]],
                bufs[c % nbuf], gsems[c % nbuf])

        gh = {c: start_g(c) for c in range(min(nbuf, n_ch))}
        for c in range(n_ch):
            gh[c].wait()
            wbh = pltpu.async_copy(
                bufs[c % nbuf], out_hbm.at[pl.ds(base + c * ch, ch)],
                wsems[c % nbuf])
            wbh.wait()
            if c + nbuf < n_ch:
                gh[c + nbuf] = start_g(c + nbuf)

    return k(table, idx)


def _fold_body(syn_W2, com_W2, out_Wp, syn_b2, com_b2, out_bp,
               fold_s, fold_c, cbias):
    d = syn_W2.shape[1]
    wo_s = out_Wp[:d, :]
    wo_c = out_Wp[d:, :]
    fold_s[...] = jnp.dot(syn_W2[...], wo_s, preferred_element_type=jnp.float32)
    fold_c[...] = jnp.dot(com_W2[...], wo_c, preferred_element_type=jnp.float32)
    cbias[...] = (jnp.dot(syn_b2[...], wo_s, preferred_element_type=jnp.float32)
                  + jnp.dot(com_b2[...], wo_c, preferred_element_type=jnp.float32)
                  + out_bp[...])


SB = 16


def _main_body(x_ref, a1_ref, a2_ref, w1s_ref, w1c_ref, b1s_ref, b1c_ref,
               folds_ref, foldc_ref, cbias_ref, out_ref):
    bf = jnp.bfloat16
    l, d = x_ref.shape[1], x_ref.shape[2]
    x2 = x_ref[...].reshape(SB * l, d).astype(bf)
    ps2 = jnp.dot(x2, w1s_ref[...],
                  preferred_element_type=jnp.float32).astype(bf)
    pc2 = jnp.dot(x2, w1c_ref[...],
                  preferred_element_type=jnp.float32).astype(bf)
    for s in range(SB):
        a1 = a1_ref[s].astype(bf)
        a2 = a2_ref[s].astype(bf)
        ps = ps2[s * l:(s + 1) * l]
        pc = pc2[s * l:(s + 1) * l]
        hs = jnp.maximum(
            jnp.dot(a1, ps, preferred_element_type=jnp.float32).astype(bf)
            + b1s_ref[...], jnp.array(0, bf))
        hc = jnp.maximum(
            jnp.dot(a2, pc, preferred_element_type=jnp.float32).astype(bf)
            + b1c_ref[...], jnp.array(0, bf))
        qs = jnp.dot(hs, folds_ref[...], preferred_element_type=jnp.float32)
        qc = jnp.dot(hc, foldc_ref[...], preferred_element_type=jnp.float32)
        out_ref[s] = (
            jnp.dot(a1, qs.astype(bf), preferred_element_type=jnp.float32)
            + jnp.dot(a2, qc.astype(bf), preferred_element_type=jnp.float32)
            + cbias_ref[...])


def kernel(inputs, adj1, adj2, adj3, table, syn_W1, syn_b1, syn_W2, syn_b2,
           com_W1, com_b1, com_W2, com_b2, out_W, out_b):
    b, l = inputs.shape
    d, h = syn_W1.shape
    o = out_W.shape[1]

    idx = inputs.reshape(b * l).astype(jnp.int32)
    nchunk = 1
    bc = b // nchunk
    xs = [_sc_gather(table, idx[k * bc * l:(k + 1) * bc * l]).reshape(bc, l, d)
          for k in range(nchunk)]

    out_Wp = jnp.zeros((2 * d, OP), jnp.float32).at[:, :o].set(out_W)
    out_bp = jnp.zeros((1, OP), jnp.float32).at[:, :o].set(out_b)

    fold_s, fold_c, cbias = pl.pallas_call(
        _fold_body,
        out_shape=[
            jax.ShapeDtypeStruct((h, OP), jnp.float32),
            jax.ShapeDtypeStruct((h, OP), jnp.float32),
            jax.ShapeDtypeStruct((1, OP), jnp.float32),
        ],
    )(syn_W2, com_W2, out_Wp, syn_b2.reshape(1, d), com_b2.reshape(1, d),
      out_bp)

    full = lambda shape: pl.BlockSpec(shape, lambda i: (0,) * len(shape))
    main = pl.pallas_call(
        _main_body,
        grid=(bc // SB,),
        in_specs=[
            pl.BlockSpec((SB, l, d), lambda i: (i, 0, 0)),
            pl.BlockSpec((SB, l, l), lambda i: (i, 0, 0)),
            pl.BlockSpec((SB, l, l), lambda i: (i, 0, 0)),
            full((d, h)),
            full((d, h)),
            full((1, h)),
            full((1, h)),
            full((h, OP)),
            full((h, OP)),
            full((1, OP)),
        ],
        out_specs=pl.BlockSpec((SB, l, OP), lambda i: (i, 0, 0)),
        out_shape=jax.ShapeDtypeStruct((bc, l, OP), jnp.float32),
        compiler_params=pltpu.CompilerParams(
            dimension_semantics=("arbitrary",)),
    )
    w1s = syn_W1.astype(jnp.bfloat16)
    w1c = com_W1.astype(jnp.bfloat16)
    fs = fold_s.astype(jnp.bfloat16)
    fc = fold_c.astype(jnp.bfloat16)
    outs = [
        main(xs[k], adj1[k * bc:(k + 1) * bc], adj2[k * bc:(k + 1) * bc],
             w1s, w1c, syn_b1.reshape(1, h).astype(jnp.bfloat16),
             com_b1.reshape(1, h).astype(jnp.bfloat16),
             fs, fc, cbias)
        for k in range(nchunk)
    ]
    return jnp.concatenate(outs, axis=0)[:, :, :o]


# speedup vs baseline: 1.0053x; 1.0028x over previous
import functools

import jax
import jax.numpy as jnp
from jax import lax
from jax.experimental import pallas as pl
from jax.experimental.pallas import tpu as pltpu
from jax.experimental.pallas import tpu_sc as plsc

OP = 3


def _sc_gather(table, idx):
    info = plsc.get_sparse_core_info()
    nw = info.num_cores * info.num_subcores
    n, d = idx.shape[0], table.shape[1]
    n_per_w = n // nw
    ch = 64
    n_ch = n_per_w // ch
    nbuf = 3
    mesh = plsc.VectorSubcoreMesh(core_axis_name="c", subcore_axis_name="s")

    @functools.partial(
        pl.kernel,
        mesh=mesh,
        out_type=jax.ShapeDtypeStruct((n, d), jnp.float32),
        scratch_types=[pltpu.VMEM((n_per_w,), jnp.int32)]
        + [pltpu.VMEM((ch, d), jnp.float32)] * nbuf
        + [pltpu.SemaphoreType.DMA] * (2 * nbuf),
    )
    def k(table_hbm, idx_hbm, out_hbm, idx_v, *bufsem):
        bufs = bufsem[:nbuf]
        gsems = bufsem[nbuf:2 * nbuf]
        wsems = bufsem[2 * nbuf:]
        wid = lax.axis_index("s") * info.num_cores + lax.axis_index("c")
        base = wid * n_per_w
        pltpu.sync_copy(idx_hbm.at[pl.ds(base, n_per_w)], idx_v)

        def start_g(c):
            return pltpu.async_copy(
                table_hbm.at[idx_v.at[pl.ds(c * ch, ch)]],
                bufs[c % nbuf], gsems[c % nbuf])

        gh = {c: start_g(c) for c in range(min(nbuf, n_ch))}
        for c in range(n_ch):
            gh[c].wait()
            wbh = pltpu.async_copy(
                bufs[c % nbuf], out_hbm.at[pl.ds(base + c * ch, ch)],
                wsems[c % nbuf])
            wbh.wait()
            if c + nbuf < n_ch:
                gh[c + nbuf] = start_g(c + nbuf)

    return k(table, idx)


def _fold_body(syn_W2, com_W2, out_Wp, syn_b2, com_b2, out_bp,
               fold_s, fold_c, cbias):
    d = syn_W2.shape[1]
    wo_s = out_Wp[:d, :]
    wo_c = out_Wp[d:, :]
    fold_s[...] = jnp.dot(syn_W2[...], wo_s, preferred_element_type=jnp.float32)
    fold_c[...] = jnp.dot(com_W2[...], wo_c, preferred_element_type=jnp.float32)
    cbias[...] = (jnp.dot(syn_b2[...], wo_s, preferred_element_type=jnp.float32)
                  + jnp.dot(com_b2[...], wo_c, preferred_element_type=jnp.float32)
                  + out_bp[...])


SB = 16


def _main_body(x_ref, a1_ref, a2_ref, w1s_ref, w1c_ref, b1s_ref, b1c_ref,
               folds_ref, foldc_ref, cbias_ref, out_ref):
    bf = jnp.bfloat16
    l, d = x_ref.shape[1], x_ref.shape[2]
    x2 = x_ref[...].reshape(SB * l, d).astype(bf)
    ps2 = jnp.dot(x2, w1s_ref[...],
                  preferred_element_type=jnp.float32).astype(bf)
    pc2 = jnp.dot(x2, w1c_ref[...],
                  preferred_element_type=jnp.float32).astype(bf)
    for s in range(SB):
        a1 = a1_ref[s].astype(bf)
        a2 = a2_ref[s].astype(bf)
        ps = ps2[s * l:(s + 1) * l]
        pc = pc2[s * l:(s + 1) * l]
        hs = jnp.maximum(
            jnp.dot(a1, ps, preferred_element_type=jnp.float32).astype(bf)
            + b1s_ref[...], jnp.array(0, bf))
        hc = jnp.maximum(
            jnp.dot(a2, pc, preferred_element_type=jnp.float32).astype(bf)
            + b1c_ref[...], jnp.array(0, bf))
        qs = jnp.dot(hs, folds_ref[...], preferred_element_type=jnp.float32)
        qc = jnp.dot(hc, foldc_ref[...], preferred_element_type=jnp.float32)
        out_ref[s] = (
            jnp.dot(a1, qs.astype(bf), preferred_element_type=jnp.float32)
            + jnp.dot(a2, qc.astype(bf), preferred_element_type=jnp.float32)
            + cbias_ref[...])


def kernel(inputs, adj1, adj2, adj3, table, syn_W1, syn_b1, syn_W2, syn_b2,
           com_W1, com_b1, com_W2, com_b2, out_W, out_b):
    b, l = inputs.shape
    d, h = syn_W1.shape
    o = out_W.shape[1]

    idx = inputs.reshape(b * l).astype(jnp.int32)
    nchunk = 1
    bc = b // nchunk
    xs = [_sc_gather(table, idx[k * bc * l:(k + 1) * bc * l]).reshape(bc, l, d)
          for k in range(nchunk)]

    out_Wp = jnp.zeros((2 * d, OP), jnp.float32).at[:, :o].set(out_W)
    out_bp = jnp.zeros((1, OP), jnp.float32).at[:, :o].set(out_b)

    fold_s, fold_c, cbias = pl.pallas_call(
        _fold_body,
        out_shape=[
            jax.ShapeDtypeStruct((h, OP), jnp.float32),
            jax.ShapeDtypeStruct((h, OP), jnp.float32),
            jax.ShapeDtypeStruct((1, OP), jnp.float32),
        ],
    )(syn_W2, com_W2, out_Wp, syn_b2.reshape(1, d), com_b2.reshape(1, d),
      out_bp)

    full = lambda shape: pl.BlockSpec(shape, lambda i: (0,) * len(shape))
    main = pl.pallas_call(
        _main_body,
        grid=(bc // SB,),
        in_specs=[
            pl.BlockSpec((SB, l, d), lambda i: (i, 0, 0)),
            pl.BlockSpec((SB, l, l), lambda i: (i, 0, 0)),
            pl.BlockSpec((SB, l, l), lambda i: (i, 0, 0)),
            full((d, h)),
            full((d, h)),
            full((1, h)),
            full((1, h)),
            full((h, OP)),
            full((h, OP)),
            full((1, OP)),
        ],
        out_specs=pl.BlockSpec((SB, l, OP), lambda i: (i, 0, 0)),
        out_shape=jax.ShapeDtypeStruct((bc, l, OP), jnp.float32),
        compiler_params=pltpu.CompilerParams(
            dimension_semantics=("arbitrary",)),
    )
    w1s = syn_W1.astype(jnp.bfloat16)
    w1c = com_W1.astype(jnp.bfloat16)
    fs = fold_s.astype(jnp.bfloat16)
    fc = fold_c.astype(jnp.bfloat16)
    outs = [
        main(xs[k], adj1[k * bc:(k + 1) * bc], adj2[k * bc:(k + 1) * bc],
             w1s, w1c, syn_b1.reshape(1, h).astype(jnp.bfloat16),
             com_b1.reshape(1, h).astype(jnp.bfloat16),
             fs, fc, cbias)
        for k in range(nchunk)
    ]
    out = outs[0] if nchunk == 1 else jnp.concatenate(outs, axis=0)
    return out if OP == o else out[:, :, :o]
